# baseline (speedup 1.0000x reference)
import functools

import jax
import jax.numpy as jnp
from jax import lax
from jax.experimental import pallas as pl
from jax.experimental.pallas import tpu as pltpu
from jax.experimental.pallas import tpu_sc as plsc

NC = 2
NS = 16
CH = 64


def _make_sc_agg(n_nodes, d, tch0, tch1, stripe, with_cnt):
    acc_rows = stripe * NS
    zlen = ((stripe + 15) // 16) * 16
    n_full = stripe // CH
    rem = stripe - n_full * CH
    nb = 4
    la = 2
    sb = 32
    assert tch0 % sb == 0 and tch1 % sb == 0 and sb % nb == 0
    adt = jnp.float32

    nco = 1 if tch1 == 0 else NC
    mesh = plsc.VectorSubcoreMesh(
        core_axis_name="c", subcore_axis_name="s", num_cores=nco,
        num_subcores=NS,
    )
    out_type = [jax.ShapeDtypeStruct((nco, acc_rows, d), adt)]
    scratch = [
        pltpu.VMEM((sb, CH), jnp.int32),
        pltpu.VMEM((sb, CH), jnp.int32),
        pltpu.VMEM((nb, CH, d), adt),
        pltpu.VMEM_SHARED((acc_rows, d), adt),
    ] + [pltpu.SemaphoreType.DMA] * (2 * nb)
    if with_cnt:
        out_type.append(jax.ShapeDtypeStruct((nco * acc_rows,), jnp.float32))
        scratch += [
            pltpu.VMEM((CH,), jnp.float32),
            pltpu.VMEM((zlen,), jnp.float32),
            pltpu.VMEM_SHARED((acc_rows,), jnp.float32),
            pltpu.SemaphoreType.DMA,
        ]

    def body(src_hbm, dst_hbm, y_hbm, *rest):
        if with_cnt:
            (out_acc, out_cnt, src_v, dst_v, rowsb_v, acc_sh,
             *sems, ones_v, zc_v, cnt_sh, csem) = rest
        else:
            out_acc, src_v, dst_v, rowsb_v, acc_sh, *sems = rest
        gsems = sems[:nb]
        ssems = sems[nb:2 * nb]
        rows_v = rowsb_v.at[0]
        c = lax.axis_index("c")
        s = lax.axis_index("s")
        tch_c = jnp.where(c == 0, tch0, tch1)
        tile_row0 = c * (NS * tch0) + s * tch_c
        base = s * stripe

        zero16 = jnp.zeros((16,), jnp.float32)

        def zrow(i, _):
            rows_v[i // (d // 16), pl.ds((i % (d // 16)) * 16, 16)] = zero16
            return 0

        lax.fori_loop(0, CH * (d // 16), zrow, 0)

        def core_guard(fn):
            if tch1 == 0:
                pl.when(c == 0)(fn)
            else:
                fn()

        def zero_acc():
            def zacc(j, _):
                pltpu.sync_copy(rows_v, acc_sh.at[pl.ds(base + j * CH, CH)])
                return 0

            lax.fori_loop(0, n_full, zacc, 0)
            if rem:
                pltpu.sync_copy(rows_v.at[pl.ds(0, rem)],
                                acc_sh.at[pl.ds(base + n_full * CH, rem)])

        core_guard(zero_acc)

        if with_cnt:
            one16 = jnp.ones((16,), jnp.float32)

            def zone(k, _):
                ones_v[pl.ds(k * 16, 16)] = one16
                return 0

            lax.fori_loop(0, CH // 16, zone, 0)

            def zzc(k, _):
                zc_v[pl.ds(k * 16, 16)] = zero16
                return 0

            lax.fori_loop(0, zlen // 16, zzc, 0)

            def zero_cnt():
                pltpu.sync_copy(zc_v.at[pl.ds(0, stripe)],
                                cnt_sh.at[pl.ds(base, stripe)])

            core_guard(zero_cnt)

        plsc.subcore_barrier()

        def g_issue(j, b):
            pltpu.async_copy(y_hbm.at[src_v.at[j]], rowsb_v.at[b], gsems[b])

        def g_wait(j, b):
            pltpu.make_async_copy(y_hbm.at[src_v.at[j]], rowsb_v.at[b],
                                  gsems[b]).wait()

        def s_issue(j, b):
            pltpu.async_copy(rowsb_v.at[b], acc_sh.at[dst_v.at[j]], ssems[b],
                             add=True)

        def s_wait(b):
            pltpu.make_async_copy(rowsb_v.at[b], acc_sh.at[dst_v.at[0]],
                                  ssems[b]).wait()

        def sblock(u, _):
            row0 = tile_row0 + u * sb
            pltpu.sync_copy(src_hbm.at[pl.ds(row0, sb)], src_v)
            pltpu.sync_copy(dst_hbm.at[pl.ds(row0, sb)], dst_v)

            for b in range(la):
                g_issue(b, b)

            def outer(t, _):
                for b in range(nb):
                    j = t * nb + b
                    jn = j + la
                    bn = (b + la) % nb
                    if b < nb - la:
                        @pl.when(jn < sb)
                        def _():
                            @pl.when(t >= 1)
                            def _():
                                s_wait(bn)
                            g_issue(jn, bn)
                    else:
                        @pl.when(jn < sb)
                        def _():
                            s_wait(bn)
                            g_issue(jn, bn)
                    g_wait(j, b)
                    s_issue(j, b)
                    if with_cnt:
                        @pl.when(j >= 1)
                        def _():
                            pltpu.make_async_copy(
                                ones_v, cnt_sh.at[dst_v.at[0]], csem).wait()
                        pltpu.async_copy(ones_v, cnt_sh.at[dst_v.at[j]], csem,
                                         add=True)
                return 0

            lax.fori_loop(0, sb // nb, outer, 0)
            for b in range(nb):
                s_wait(b)
            if with_cnt:
                pltpu.make_async_copy(ones_v, cnt_sh.at[dst_v.at[0]],
                                      csem).wait()
            return 0

        lax.fori_loop(0, tch_c // sb, sblock, 0)

        plsc.subcore_barrier()

        def writeback():
            def wb(j, _):
                pltpu.sync_copy(acc_sh.at[pl.ds(base + j * CH, CH)], rows_v)
                pltpu.sync_copy(rows_v,
                                out_acc.at[c, pl.ds(base + j * CH, CH)])
                return 0

            lax.fori_loop(0, n_full, wb, 0)
            if rem:
                pltpu.sync_copy(acc_sh.at[pl.ds(base + n_full * CH, rem)],
                                rows_v.at[pl.ds(0, rem)])
                pltpu.sync_copy(rows_v.at[pl.ds(0, rem)],
                                out_acc.at[c, pl.ds(base + n_full * CH, rem)])
            if with_cnt:
                pltpu.sync_copy(cnt_sh.at[pl.ds(base, stripe)],
                                zc_v.at[pl.ds(0, stripe)])
                pltpu.sync_copy(zc_v.at[pl.ds(0, stripe)],
                                out_cnt.at[pl.ds(c * acc_rows + base, stripe)])

        core_guard(writeback)

    return pl.kernel(body, out_type=out_type, mesh=mesh, scratch_types=scratch)


def _mm_t(a, w):
    return lax.dot_general(a, w, (((1,), (1,)), ((), ())),
                           preferred_element_type=jnp.float32)


def _tc_in(x, w_l, w_r, b):
    def body(x_ref, wl_ref, wr_ref, b_ref, y_ref, z_ref):
        xv = x_ref[...]
        y_ref[...] = _mm_t(xv, wl_ref[...])
        z_ref[...] = _mm_t(xv, wr_ref[...]) + b_ref[...]

    n, d = x.shape
    return pl.pallas_call(
        body,
        out_shape=[jax.ShapeDtypeStruct((n, d), jnp.float32),
                   jax.ShapeDtypeStruct((n, d), jnp.float32)],
    )(x, w_l, w_r, b)


def _tc_mid(ps, cs, z1, w_l, w_r, b):
    k = len(ps)

    def body(*refs):
        p_refs, c_refs = refs[:k], refs[k:2 * k]
        z1_ref, wl_ref, wr_ref, b_ref, y_ref, z_ref, cnt_ref = refs[2 * k:]
        cnt = jnp.maximum(sum(r[...] for r in c_refs), 1.0)
        agg = sum(r[...].astype(jnp.float32) for r in p_refs)
        h = jnp.maximum(agg / cnt + z1_ref[...], 0.0)
        y_ref[...] = _mm_t(h, wl_ref[...])
        z_ref[...] = _mm_t(h, wr_ref[...]) + b_ref[...]
        cnt_ref[...] = cnt

    n, d = ps[0].shape
    return pl.pallas_call(
        body,
        out_shape=[jax.ShapeDtypeStruct((n, d), jnp.float32),
                   jax.ShapeDtypeStruct((n, d), jnp.float32),
                   jax.ShapeDtypeStruct((n, 1), jnp.float32)],
    )(*ps, *cs, z1, w_l, w_r, b)


def _tc_out(qs, cnt, z2):
    k = len(qs)

    def body(*refs):
        q_refs = refs[:k]
        cnt_ref, z2_ref, o_ref = refs[k:]
        agg = sum(r[...].astype(jnp.float32) for r in q_refs)
        o_ref[...] = agg / cnt_ref[...] + z2_ref[...]

    n, d = qs[0].shape
    return pl.pallas_call(
        body,
        out_shape=jax.ShapeDtypeStruct((n, d), jnp.float32),
    )(*qs, cnt, z2)


def kernel(x, edge_index, W1_l, b1, W1_r, W2_l, b2, W2_r):
    n, d = x.shape
    e = edge_index.shape[1]
    tp = -(-(-(-e // CH)) // NS)
    tp = ((tp + 159) // 160) * 160
    tch0 = tp * 4 // 5
    tch1 = tp - tch0
    e_pad = NS * tp * CH
    stripe = ((-(-(n + 1) // NS)) + 15) // 16 * 16

    src = edge_index[0].astype(jnp.int32)
    dst = edge_index[1].astype(jnp.int32)
    pad = e_pad - e
    src_p = jnp.concatenate([src, jnp.zeros((pad,), jnp.int32)])
    dst_p = jnp.concatenate([dst, jnp.full((pad,), n, jnp.int32)])
    src_p = src_p.reshape(NS * tp, CH)
    dst_p = dst_p.reshape(NS * tp, CH)

    sc_agg1 = _make_sc_agg(n, d, tch0, tch1, stripe, with_cnt=True)
    sc_agg2 = _make_sc_agg(n, d, tch0, tch1, stripe, with_cnt=False)

    nco = 1 if tch1 == 0 else NC
    y1, z1 = _tc_in(x, W1_l, W1_r, b1.reshape(1, d))
    p, cpart = sc_agg1(src_p, dst_p, y1)
    cpart = cpart.reshape(nco, stripe * NS)
    y2, z2, cnt = _tc_mid([p[i, :n] for i in range(nco)],
                          [cpart[i, :n, None] for i in range(nco)],
                          z1, W2_l, W2_r, b2.reshape(1, d))
    q = sc_agg2(src_p, dst_p, y2)
    if isinstance(q, (list, tuple)):
        q = q[0]
    return _tc_out([q[i, :n] for i in range(nco)], cnt, z2)


# speedup vs baseline: 1.1586x; 1.1586x over previous
import functools

import jax
import jax.numpy as jnp
from jax import lax
from jax.experimental import pallas as pl
from jax.experimental.pallas import tpu as pltpu
from jax.experimental.pallas import tpu_sc as plsc

NC = 2
NS = 16
CH = 128


def _make_sc_agg(n_nodes, d, tch0, tch1, stripe, with_cnt):
    acc_rows = stripe * NS
    zlen = ((stripe + 15) // 16) * 16
    n_full = stripe // CH
    rem = stripe - n_full * CH
    nb = 2
    la = 1
    sb = 16
    assert tch0 % sb == 0 and tch1 % sb == 0 and sb % nb == 0
    adt = jnp.float32

    nco = 1 if tch1 == 0 else NC
    mesh = plsc.VectorSubcoreMesh(
        core_axis_name="c", subcore_axis_name="s", num_cores=nco,
        num_subcores=NS,
    )
    out_type = [jax.ShapeDtypeStruct((nco, acc_rows, d), adt)]
    scratch = [
        pltpu.VMEM((sb, CH), jnp.int32),
        pltpu.VMEM((sb, CH), jnp.int32),
        pltpu.VMEM((nb, CH, d), adt),
        pltpu.VMEM_SHARED((acc_rows, d), adt),
    ] + [pltpu.SemaphoreType.DMA] * (2 * nb)
    if with_cnt:
        out_type.append(jax.ShapeDtypeStruct((nco * acc_rows,), jnp.float32))
        scratch += [
            pltpu.VMEM((CH,), jnp.float32),
            pltpu.VMEM((zlen,), jnp.float32),
            pltpu.VMEM_SHARED((acc_rows,), jnp.float32),
            pltpu.SemaphoreType.DMA,
        ]

    def body(src_hbm, dst_hbm, y_hbm, *rest):
        if with_cnt:
            (out_acc, out_cnt, src_v, dst_v, rowsb_v, acc_sh,
             *sems, ones_v, zc_v, cnt_sh, csem) = rest
        else:
            out_acc, src_v, dst_v, rowsb_v, acc_sh, *sems = rest
        gsems = sems[:nb]
        ssems = sems[nb:2 * nb]
        rows_v = rowsb_v.at[0]
        c = lax.axis_index("c")
        s = lax.axis_index("s")
        tch_c = jnp.where(c == 0, tch0, tch1)
        tile_row0 = c * (NS * tch0) + s * tch_c
        base = s * stripe

        zero16 = jnp.zeros((16,), jnp.float32)

        def zrow(i, _):
            rows_v[i // (d // 16), pl.ds((i % (d // 16)) * 16, 16)] = zero16
            return 0

        lax.fori_loop(0, CH * (d // 16), zrow, 0)

        def core_guard(fn):
            if tch1 == 0:
                pl.when(c == 0)(fn)
            else:
                fn()

        def zero_acc():
            def zacc(j, _):
                pltpu.sync_copy(rows_v, acc_sh.at[pl.ds(base + j * CH, CH)])
                return 0

            lax.fori_loop(0, n_full, zacc, 0)
            if rem:
                pltpu.sync_copy(rows_v.at[pl.ds(0, rem)],
                                acc_sh.at[pl.ds(base + n_full * CH, rem)])

        core_guard(zero_acc)

        if with_cnt:
            one16 = jnp.ones((16,), jnp.float32)

            def zone(k, _):
                ones_v[pl.ds(k * 16, 16)] = one16
                return 0

            lax.fori_loop(0, CH // 16, zone, 0)

            def zzc(k, _):
                zc_v[pl.ds(k * 16, 16)] = zero16
                return 0

            lax.fori_loop(0, zlen // 16, zzc, 0)

            def zero_cnt():
                pltpu.sync_copy(zc_v.at[pl.ds(0, stripe)],
                                cnt_sh.at[pl.ds(base, stripe)])

            core_guard(zero_cnt)

        plsc.subcore_barrier()

        def g_issue(j, b):
            pltpu.async_copy(y_hbm.at[src_v.at[j]], rowsb_v.at[b], gsems[b])

        def g_wait(j, b):
            pltpu.make_async_copy(y_hbm.at[src_v.at[j]], rowsb_v.at[b],
                                  gsems[b]).wait()

        def s_issue(j, b):
            pltpu.async_copy(rowsb_v.at[b], acc_sh.at[dst_v.at[j]], ssems[b],
                             add=True)

        def s_wait(b):
            pltpu.make_async_copy(rowsb_v.at[b], acc_sh.at[dst_v.at[0]],
                                  ssems[b]).wait()

        def sblock(u, _):
            row0 = tile_row0 + u * sb
            pltpu.sync_copy(src_hbm.at[pl.ds(row0, sb)], src_v)
            pltpu.sync_copy(dst_hbm.at[pl.ds(row0, sb)], dst_v)

            for b in range(la):
                g_issue(b, b)

            def outer(t, _):
                for b in range(nb):
                    j = t * nb + b
                    jn = j + la
                    bn = (b + la) % nb
                    if b < nb - la:
                        @pl.when(jn < sb)
                        def _():
                            @pl.when(t >= 1)
                            def _():
                                s_wait(bn)
                            g_issue(jn, bn)
                    else:
                        @pl.when(jn < sb)
                        def _():
                            s_wait(bn)
                            g_issue(jn, bn)
                    g_wait(j, b)
                    s_issue(j, b)
                    if with_cnt:
                        @pl.when(j >= 1)
                        def _():
                            pltpu.make_async_copy(
                                ones_v, cnt_sh.at[dst_v.at[0]], csem).wait()
                        pltpu.async_copy(ones_v, cnt_sh.at[dst_v.at[j]], csem,
                                         add=True)
                return 0

            lax.fori_loop(0, sb // nb, outer, 0)
            for b in range(nb):
                s_wait(b)
            if with_cnt:
                pltpu.make_async_copy(ones_v, cnt_sh.at[dst_v.at[0]],
                                      csem).wait()
            return 0

        lax.fori_loop(0, tch_c // sb, sblock, 0)

        plsc.subcore_barrier()

        def writeback():
            def wb(j, _):
                pltpu.sync_copy(acc_sh.at[pl.ds(base + j * CH, CH)], rows_v)
                pltpu.sync_copy(rows_v,
                                out_acc.at[c, pl.ds(base + j * CH, CH)])
                return 0

            lax.fori_loop(0, n_full, wb, 0)
            if rem:
                pltpu.sync_copy(acc_sh.at[pl.ds(base + n_full * CH, rem)],
                                rows_v.at[pl.ds(0, rem)])
                pltpu.sync_copy(rows_v.at[pl.ds(0, rem)],
                                out_acc.at[c, pl.ds(base + n_full * CH, rem)])
            if with_cnt:
                pltpu.sync_copy(cnt_sh.at[pl.ds(base, stripe)],
                                zc_v.at[pl.ds(0, stripe)])
                pltpu.sync_copy(zc_v.at[pl.ds(0, stripe)],
                                out_cnt.at[pl.ds(c * acc_rows + base, stripe)])

        core_guard(writeback)

    return pl.kernel(body, out_type=out_type, mesh=mesh, scratch_types=scratch)


def _mm_t(a, w):
    return lax.dot_general(a, w, (((1,), (1,)), ((), ())),
                           preferred_element_type=jnp.float32)


def _tc_in(x, w_l, w_r, b):
    def body(x_ref, wl_ref, wr_ref, b_ref, y_ref, z_ref):
        xv = x_ref[...]
        y_ref[...] = _mm_t(xv, wl_ref[...])
        z_ref[...] = _mm_t(xv, wr_ref[...]) + b_ref[...]

    n, d = x.shape
    return pl.pallas_call(
        body,
        out_shape=[jax.ShapeDtypeStruct((n, d), jnp.float32),
                   jax.ShapeDtypeStruct((n, d), jnp.float32)],
    )(x, w_l, w_r, b)


def _tc_mid(ps, cs, z1, w_l, w_r, b):
    k = len(ps)

    def body(*refs):
        p_refs, c_refs = refs[:k], refs[k:2 * k]
        z1_ref, wl_ref, wr_ref, b_ref, y_ref, z_ref, cnt_ref = refs[2 * k:]
        cnt = jnp.maximum(sum(r[...] for r in c_refs), 1.0)
        agg = sum(r[...].astype(jnp.float32) for r in p_refs)
        h = jnp.maximum(agg / cnt + z1_ref[...], 0.0)
        y_ref[...] = _mm_t(h, wl_ref[...])
        z_ref[...] = _mm_t(h, wr_ref[...]) + b_ref[...]
        cnt_ref[...] = cnt

    n, d = ps[0].shape
    return pl.pallas_call(
        body,
        out_shape=[jax.ShapeDtypeStruct((n, d), jnp.float32),
                   jax.ShapeDtypeStruct((n, d), jnp.float32),
                   jax.ShapeDtypeStruct((n, 1), jnp.float32)],
    )(*ps, *cs, z1, w_l, w_r, b)


def _tc_out(qs, cnt, z2):
    k = len(qs)

    def body(*refs):
        q_refs = refs[:k]
        cnt_ref, z2_ref, o_ref = refs[k:]
        agg = sum(r[...].astype(jnp.float32) for r in q_refs)
        o_ref[...] = agg / cnt_ref[...] + z2_ref[...]

    n, d = qs[0].shape
    return pl.pallas_call(
        body,
        out_shape=jax.ShapeDtypeStruct((n, d), jnp.float32),
    )(*qs, cnt, z2)


def kernel(x, edge_index, W1_l, b1, W1_r, W2_l, b2, W2_r):
    n, d = x.shape
    e = edge_index.shape[1]
    tp = -(-(-(-e // CH)) // NS)
    tp = ((tp + 159) // 160) * 160
    tch0 = tp * 9 // 10
    tch1 = tp - tch0
    e_pad = NS * tp * CH
    stripe = ((-(-(n + 1) // NS)) + 15) // 16 * 16

    src = edge_index[0].astype(jnp.int32)
    dst = edge_index[1].astype(jnp.int32)
    pad = e_pad - e
    src_p = jnp.concatenate([src, jnp.zeros((pad,), jnp.int32)])
    dst_p = jnp.concatenate([dst, jnp.full((pad,), n, jnp.int32)])
    src_p = src_p.reshape(NS * tp, CH)
    dst_p = dst_p.reshape(NS * tp, CH)

    sc_agg1 = _make_sc_agg(n, d, tch0, tch1, stripe, with_cnt=True)
    sc_agg2 = _make_sc_agg(n, d, tch0, tch1, stripe, with_cnt=False)

    nco = 1 if tch1 == 0 else NC
    y1, z1 = _tc_in(x, W1_l, W1_r, b1.reshape(1, d))
    p, cpart = sc_agg1(src_p, dst_p, y1)
    cpart = cpart.reshape(nco, stripe * NS)
    y2, z2, cnt = _tc_mid([p[i, :n] for i in range(nco)],
                          [cpart[i, :n, None] for i in range(nco)],
                          z1, W2_l, W2_r, b2.reshape(1, d))
    q = sc_agg2(src_p, dst_p, y2)
    if isinstance(q, (list, tuple)):
        q = q[0]
    return _tc_out([q[i, :n] for i in range(nco)], cnt, z2)


# speedup vs baseline: 1.1749x; 1.0141x over previous
import functools

import jax
import jax.numpy as jnp
from jax import lax
from jax.experimental import pallas as pl
from jax.experimental.pallas import tpu as pltpu
from jax.experimental.pallas import tpu_sc as plsc

NC = 2
NS = 16
CH = 128


def _make_sc_agg(n_nodes, d, tch0, tch1, stripe, with_cnt):
    acc_rows = stripe * NS
    zlen = ((stripe + 15) // 16) * 16
    n_full = stripe // CH
    rem = stripe - n_full * CH
    nb = 2
    la = 1
    sb = 8
    assert tch0 % sb == 0 and tch1 % sb == 0 and sb % nb == 0
    adt = jnp.float32

    nco = 1 if tch1 == 0 else NC
    mesh = plsc.VectorSubcoreMesh(
        core_axis_name="c", subcore_axis_name="s", num_cores=nco,
        num_subcores=NS,
    )
    out_type = [jax.ShapeDtypeStruct((nco, acc_rows, d), adt)]
    scratch = [
        pltpu.VMEM((sb, CH), jnp.int32),
        pltpu.VMEM((sb, CH), jnp.int32),
        pltpu.VMEM((nb, CH, d), adt),
        pltpu.VMEM_SHARED((acc_rows, d), adt),
    ] + [pltpu.SemaphoreType.DMA] * (2 * nb)
    if with_cnt:
        out_type.append(jax.ShapeDtypeStruct((nco * acc_rows,), jnp.float32))
        scratch += [
            pltpu.VMEM((CH,), jnp.float32),
            pltpu.VMEM((zlen,), jnp.float32),
            pltpu.VMEM_SHARED((acc_rows,), jnp.float32),
            pltpu.SemaphoreType.DMA,
        ]

    def body(src_hbm, dst_hbm, y_hbm, *rest):
        if with_cnt:
            (out_acc, out_cnt, src_v, dst_v, rowsb_v, acc_sh,
             *sems, ones_v, zc_v, cnt_sh, csem) = rest
        else:
            out_acc, src_v, dst_v, rowsb_v, acc_sh, *sems = rest
        gsems = sems[:nb]
        ssems = sems[nb:2 * nb]
        rows_v = rowsb_v.at[0]
        c = lax.axis_index("c")
        s = lax.axis_index("s")
        tch_c = jnp.where(c == 0, tch0, tch1)
        tile_row0 = c * (NS * tch0) + s * tch_c
        base = s * stripe

        zero16 = jnp.zeros((16,), jnp.float32)

        def zrow(i, _):
            rows_v[i // (d // 16), pl.ds((i % (d // 16)) * 16, 16)] = zero16
            return 0

        lax.fori_loop(0, CH * (d // 16), zrow, 0)

        def core_guard(fn):
            if tch1 == 0:
                pl.when(c == 0)(fn)
            else:
                fn()

        def zero_acc():
            def zacc(j, _):
                pltpu.sync_copy(rows_v, acc_sh.at[pl.ds(base + j * CH, CH)])
                return 0

            lax.fori_loop(0, n_full, zacc, 0)
            if rem:
                pltpu.sync_copy(rows_v.at[pl.ds(0, rem)],
                                acc_sh.at[pl.ds(base + n_full * CH, rem)])

        core_guard(zero_acc)

        if with_cnt:
            one16 = jnp.ones((16,), jnp.float32)

            def zone(k, _):
                ones_v[pl.ds(k * 16, 16)] = one16
                return 0

            lax.fori_loop(0, CH // 16, zone, 0)

            def zzc(k, _):
                zc_v[pl.ds(k * 16, 16)] = zero16
                return 0

            lax.fori_loop(0, zlen // 16, zzc, 0)

            def zero_cnt():
                pltpu.sync_copy(zc_v.at[pl.ds(0, stripe)],
                                cnt_sh.at[pl.ds(base, stripe)])

            core_guard(zero_cnt)

        plsc.subcore_barrier()

        def g_issue(j, b):
            pltpu.async_copy(y_hbm.at[src_v.at[j]], rowsb_v.at[b], gsems[b])

        def g_wait(j, b):
            pltpu.make_async_copy(y_hbm.at[src_v.at[j]], rowsb_v.at[b],
                                  gsems[b]).wait()

        def s_issue(j, b):
            pltpu.async_copy(rowsb_v.at[b], acc_sh.at[dst_v.at[j]], ssems[b],
                             add=True)

        def s_wait(b):
            pltpu.make_async_copy(rowsb_v.at[b], acc_sh.at[dst_v.at[0]],
                                  ssems[b]).wait()

        def sblock(u, _):
            row0 = tile_row0 + u * sb
            pltpu.sync_copy(src_hbm.at[pl.ds(row0, sb)], src_v)
            pltpu.sync_copy(dst_hbm.at[pl.ds(row0, sb)], dst_v)

            for b in range(la):
                g_issue(b, b)

            def outer(t, _):
                for b in range(nb):
                    j = t * nb + b
                    jn = j + la
                    bn = (b + la) % nb
                    if b < nb - la:
                        @pl.when(jn < sb)
                        def _():
                            @pl.when(t >= 1)
                            def _():
                                s_wait(bn)
                            g_issue(jn, bn)
                    else:
                        @pl.when(jn < sb)
                        def _():
                            s_wait(bn)
                            g_issue(jn, bn)
                    g_wait(j, b)
                    s_issue(j, b)
                    if with_cnt:
                        @pl.when(j >= 1)
                        def _():
                            pltpu.make_async_copy(
                                ones_v, cnt_sh.at[dst_v.at[0]], csem).wait()
                        pltpu.async_copy(ones_v, cnt_sh.at[dst_v.at[j]], csem,
                                         add=True)
                return 0

            lax.fori_loop(0, sb // nb, outer, 0)
            for b in range(nb):
                s_wait(b)
            if with_cnt:
                pltpu.make_async_copy(ones_v, cnt_sh.at[dst_v.at[0]],
                                      csem).wait()
            return 0

        lax.fori_loop(0, tch_c // sb, sblock, 0)

        plsc.subcore_barrier()

        def writeback():
            def wb(j, _):
                pltpu.sync_copy(acc_sh.at[pl.ds(base + j * CH, CH)], rows_v)
                pltpu.sync_copy(rows_v,
                                out_acc.at[c, pl.ds(base + j * CH, CH)])
                return 0

            lax.fori_loop(0, n_full, wb, 0)
            if rem:
                pltpu.sync_copy(acc_sh.at[pl.ds(base + n_full * CH, rem)],
                                rows_v.at[pl.ds(0, rem)])
                pltpu.sync_copy(rows_v.at[pl.ds(0, rem)],
                                out_acc.at[c, pl.ds(base + n_full * CH, rem)])
            if with_cnt:
                pltpu.sync_copy(cnt_sh.at[pl.ds(base, stripe)],
                                zc_v.at[pl.ds(0, stripe)])
                pltpu.sync_copy(zc_v.at[pl.ds(0, stripe)],
                                out_cnt.at[pl.ds(c * acc_rows + base, stripe)])

        core_guard(writeback)

    return pl.kernel(body, out_type=out_type, mesh=mesh, scratch_types=scratch)


def _mm_t(a, w):
    return lax.dot_general(a, w, (((1,), (1,)), ((), ())),
                           preferred_element_type=jnp.float32)


def _tc_in(x, w_l, w_r, b):
    def body(x_ref, wl_ref, wr_ref, b_ref, y_ref, z_ref):
        xv = x_ref[...]
        y_ref[...] = _mm_t(xv, wl_ref[...])
        z_ref[...] = _mm_t(xv, wr_ref[...]) + b_ref[...]

    n, d = x.shape
    return pl.pallas_call(
        body,
        out_shape=[jax.ShapeDtypeStruct((n, d), jnp.float32),
                   jax.ShapeDtypeStruct((n, d), jnp.float32)],
    )(x, w_l, w_r, b)


def _tc_mid(ps, cs, z1, w_l, w_r, b):
    k = len(ps)

    def body(*refs):
        p_refs, c_refs = refs[:k], refs[k:2 * k]
        z1_ref, wl_ref, wr_ref, b_ref, y_ref, z_ref, cnt_ref = refs[2 * k:]
        cnt = jnp.maximum(sum(r[...] for r in c_refs), 1.0)
        agg = sum(r[...].astype(jnp.float32) for r in p_refs)
        h = jnp.maximum(agg / cnt + z1_ref[...], 0.0)
        y_ref[...] = _mm_t(h, wl_ref[...])
        z_ref[...] = _mm_t(h, wr_ref[...]) + b_ref[...]
        cnt_ref[...] = cnt

    n, d = ps[0].shape
    return pl.pallas_call(
        body,
        out_shape=[jax.ShapeDtypeStruct((n, d), jnp.float32),
                   jax.ShapeDtypeStruct((n, d), jnp.float32),
                   jax.ShapeDtypeStruct((n, 1), jnp.float32)],
    )(*ps, *cs, z1, w_l, w_r, b)


def _tc_out(qs, cnt, z2):
    k = len(qs)

    def body(*refs):
        q_refs = refs[:k]
        cnt_ref, z2_ref, o_ref = refs[k:]
        agg = sum(r[...].astype(jnp.float32) for r in q_refs)
        o_ref[...] = agg / cnt_ref[...] + z2_ref[...]

    n, d = qs[0].shape
    return pl.pallas_call(
        body,
        out_shape=jax.ShapeDtypeStruct((n, d), jnp.float32),
    )(*qs, cnt, z2)


def kernel(x, edge_index, W1_l, b1, W1_r, W2_l, b2, W2_r):
    n, d = x.shape
    e = edge_index.shape[1]
    tp = -(-(-(-e // CH)) // NS)
    tp = ((tp + 159) // 160) * 160
    tch0 = tp * 19 // 20
    tch1 = tp - tch0
    e_pad = NS * tp * CH
    stripe = ((-(-(n + 1) // NS)) + 15) // 16 * 16

    src = edge_index[0].astype(jnp.int32)
    dst = edge_index[1].astype(jnp.int32)
    pad = e_pad - e
    src_p = jnp.concatenate([src, jnp.zeros((pad,), jnp.int32)])
    dst_p = jnp.concatenate([dst, jnp.full((pad,), n, jnp.int32)])
    src_p = src_p.reshape(NS * tp, CH)
    dst_p = dst_p.reshape(NS * tp, CH)

    sc_agg1 = _make_sc_agg(n, d, tch0, tch1, stripe, with_cnt=True)
    sc_agg2 = _make_sc_agg(n, d, tch0, tch1, stripe, with_cnt=False)

    nco = 1 if tch1 == 0 else NC
    y1, z1 = _tc_in(x, W1_l, W1_r, b1.reshape(1, d))
    p, cpart = sc_agg1(src_p, dst_p, y1)
    cpart = cpart.reshape(nco, stripe * NS)
    y2, z2, cnt = _tc_mid([p[i, :n] for i in range(nco)],
                          [cpart[i, :n, None] for i in range(nco)],
                          z1, W2_l, W2_r, b2.reshape(1, d))
    q = sc_agg2(src_p, dst_p, y2)
    if isinstance(q, (list, tuple)):
        q = q[0]
    return _tc_out([q[i, :n] for i in range(nco)], cnt, z2)
